# Initial kernel scaffold; baseline (speedup 1.0000x reference)
import jax
import jax.numpy as jnp
from jax.experimental import pallas as pl


def kernel(inputs, hidden_state, adj_mx, W_ru_0, b_ru_0, W_c_0, b_c_0, W_ru_1, b_ru_1, W_c_1, b_c_1, W_proj, b_proj):
    raise NotImplementedError("write your pallas kernel here")


# speedup vs baseline: 2.1246x; 2.1246x over previous
import jax
import jax.numpy as jnp
from jax.experimental import pallas as pl

N = 512
U = 64
B = 64
NK = 3


def _prep_kernel(adj_ref, adjt_ref, x_ref, sup_ref, a1_ref, a2_ref):
    a = jnp.maximum(adj_ref[...], adjt_ref[...])
    d_col = jnp.sum(a, axis=1, keepdims=True)
    d_row = jnp.sum(a, axis=0, keepdims=True)
    inv_c = jnp.where(d_col > 0, 1.0 / jnp.sqrt(d_col), 0.0)
    inv_r = jnp.where(d_row > 0, 1.0 / jnp.sqrt(d_row), 0.0)
    sup = -(inv_c * a) * inv_r
    sup_ref[...] = sup
    x0 = x_ref[...]
    a1 = jnp.dot(sup, x0, preferred_element_type=jnp.float32)
    a1_ref[...] = a1
    a2_ref[...] = 2.0 * jnp.dot(sup, a1, preferred_element_type=jnp.float32) - x0


def _main_kernel(sup_ref, a0_ref, a1_ref, a2_ref, h0_ref, h1_ref,
                 wa_ru0_ref, wh_ru0_ref, b_ru0_ref,
                 wa_c0_ref, wh_c0_ref, b_c0_ref,
                 wg_ru1_ref, wk_ru1_ref, b_ru1_ref,
                 wg_c1_ref, wk_c1_ref, b_c1_ref,
                 wp_ref, bp_ref,
                 out_ref, h0o_ref, h1o_ref):
    f32 = jnp.float32
    S = sup_ref[...]
    a0 = a0_ref[0]
    a1 = a1_ref[0]
    a2 = a2_ref[0]

    H0 = h0_ref[0]
    ru = (b_ru0_ref[...]
          + a0 * wa_ru0_ref[0:1, :] + a1 * wa_ru0_ref[1:2, :] + a2 * wa_ru0_ref[2:3, :])
    ru += jnp.dot(H0, wh_ru0_ref[0], preferred_element_type=f32)
    H1 = jnp.dot(S, H0, preferred_element_type=f32)
    ru += jnp.dot(H1, wh_ru0_ref[1], preferred_element_type=f32)
    H2 = 2.0 * jnp.dot(S, H1, preferred_element_type=f32) - H0
    ru += jnp.dot(H2, wh_ru0_ref[2], preferred_element_type=f32)
    val = jax.nn.sigmoid(ru)
    r = val[:, :U]
    u = val[:, U:]
    rh = r * H0
    c = (b_c0_ref[...]
         + a0 * wa_c0_ref[0:1, :] + a1 * wa_c0_ref[1:2, :] + a2 * wa_c0_ref[2:3, :])
    c += jnp.dot(rh, wh_c0_ref[0], preferred_element_type=f32)
    R1 = jnp.dot(S, rh, preferred_element_type=f32)
    c += jnp.dot(R1, wh_c0_ref[1], preferred_element_type=f32)
    R2 = 2.0 * jnp.dot(S, R1, preferred_element_type=f32) - rh
    c += jnp.dot(R2, wh_c0_ref[2], preferred_element_type=f32)
    c = jnp.tanh(c)
    h0n = u * H0 + (1.0 - u) * c
    h0o_ref[0] = h0n

    K0 = h1_ref[0]
    ru1 = (b_ru1_ref[...]
           + jnp.dot(h0n, wg_ru1_ref[0], preferred_element_type=f32)
           + jnp.dot(K0, wk_ru1_ref[0], preferred_element_type=f32))
    G1 = jnp.dot(S, h0n, preferred_element_type=f32)
    ru1 += jnp.dot(G1, wg_ru1_ref[1], preferred_element_type=f32)
    K1 = jnp.dot(S, K0, preferred_element_type=f32)
    ru1 += jnp.dot(K1, wk_ru1_ref[1], preferred_element_type=f32)
    G2 = 2.0 * jnp.dot(S, G1, preferred_element_type=f32) - h0n
    ru1 += jnp.dot(G2, wg_ru1_ref[2], preferred_element_type=f32)
    K2 = 2.0 * jnp.dot(S, K1, preferred_element_type=f32) - K0
    ru1 += jnp.dot(K2, wk_ru1_ref[2], preferred_element_type=f32)
    v1 = jax.nn.sigmoid(ru1)
    r1 = v1[:, :U]
    u1 = v1[:, U:]
    rh1 = r1 * K0
    c1 = (b_c1_ref[...]
          + jnp.dot(h0n, wg_c1_ref[0], preferred_element_type=f32)
          + jnp.dot(G1, wg_c1_ref[1], preferred_element_type=f32)
          + jnp.dot(G2, wg_c1_ref[2], preferred_element_type=f32)
          + jnp.dot(rh1, wk_c1_ref[0], preferred_element_type=f32))
    Q1 = jnp.dot(S, rh1, preferred_element_type=f32)
    c1 += jnp.dot(Q1, wk_c1_ref[1], preferred_element_type=f32)
    Q2 = 2.0 * jnp.dot(S, Q1, preferred_element_type=f32) - rh1
    c1 += jnp.dot(Q2, wk_c1_ref[2], preferred_element_type=f32)
    c1 = jnp.tanh(c1)
    h1n = u1 * K0 + (1.0 - u1) * c1
    h1o_ref[0] = h1n
    out_ref[0] = jnp.dot(h1n, wp_ref[...], preferred_element_type=f32) + bp_ref[...]


def kernel(inputs, hidden_state, adj_mx, W_ru_0, b_ru_0, W_c_0, b_c_0,
           W_ru_1, b_ru_1, W_c_1, b_c_1, W_proj, b_proj):
    f32 = jnp.float32
    inp_t = inputs.T
    hs = hidden_state.reshape(2, B, N, U)

    wru0 = W_ru_0.reshape(U + 1, NK, 2 * U)
    wa_ru0 = wru0[0]
    wh_ru0 = jnp.transpose(wru0[1:], (1, 0, 2))
    wc0 = W_c_0.reshape(U + 1, NK, U)
    wa_c0 = wc0[0]
    wh_c0 = jnp.transpose(wc0[1:], (1, 0, 2))
    wru1 = W_ru_1.reshape(2 * U, NK, 2 * U)
    wg_ru1 = jnp.transpose(wru1[:U], (1, 0, 2))
    wk_ru1 = jnp.transpose(wru1[U:], (1, 0, 2))
    wc1 = W_c_1.reshape(2 * U, NK, U)
    wg_c1 = jnp.transpose(wc1[:U], (1, 0, 2))
    wk_c1 = jnp.transpose(wc1[U:], (1, 0, 2))

    b_ru0 = b_ru_0.reshape(1, 2 * U)
    b_c0 = b_c_0.reshape(1, U)
    b_ru1 = b_ru_1.reshape(1, 2 * U)
    b_c1 = b_c_1.reshape(1, U)
    bp = b_proj.reshape(1, 1)

    support, A1, A2 = pl.pallas_call(
        _prep_kernel,
        out_shape=[
            jax.ShapeDtypeStruct((N, N), f32),
            jax.ShapeDtypeStruct((N, B), f32),
            jax.ShapeDtypeStruct((N, B), f32),
        ],
    )(adj_mx, adj_mx.T, inp_t)

    a0_3 = inputs.reshape(B, N, 1)
    a1_3 = A1.T.reshape(B, N, 1)
    a2_3 = A2.T.reshape(B, N, 1)

    full = lambda shape: pl.BlockSpec(shape, lambda b: tuple(0 for _ in shape))
    col = pl.BlockSpec((1, N, 1), lambda b: (b, 0, 0))
    hblk = pl.BlockSpec((1, N, U), lambda b: (b, 0, 0))

    out_3, h0o, h1o = pl.pallas_call(
        _main_kernel,
        grid=(B,),
        in_specs=[
            full((N, N)), col, col, col, hblk, hblk,
            full((NK, 2 * U)), full((NK, U, 2 * U)), full((1, 2 * U)),
            full((NK, U)), full((NK, U, U)), full((1, U)),
            full((NK, U, 2 * U)), full((NK, U, 2 * U)), full((1, 2 * U)),
            full((NK, U, U)), full((NK, U, U)), full((1, U)),
            full((U, 1)), full((1, 1)),
        ],
        out_specs=[col, hblk, hblk],
        out_shape=[
            jax.ShapeDtypeStruct((B, N, 1), f32),
            jax.ShapeDtypeStruct((B, N, U), f32),
            jax.ShapeDtypeStruct((B, N, U), f32),
        ],
    )(support, a0_3, a1_3, a2_3, hs[0], hs[1],
      wa_ru0, wh_ru0, b_ru0, wa_c0, wh_c0, b_c0,
      wg_ru1, wk_ru1, b_ru1, wg_c1, wk_c1, b_c1,
      W_proj, bp)

    out = out_3.reshape(B, N)
    hidden = jnp.stack([h0o.reshape(B, N * U), h1o.reshape(B, N * U)])
    return (out, hidden)


# speedup vs baseline: 3.2238x; 1.5174x over previous
import jax
import jax.numpy as jnp
from jax.experimental import pallas as pl

N = 512
U = 64
B = 64
NK = 3


def _prep_kernel(adj_ref, adjt_ref, x_ref, sup_ref, a1_ref, a2_ref):
    a = jnp.maximum(adj_ref[...], adjt_ref[...])
    d_col = jnp.sum(a, axis=1, keepdims=True)
    d_row = jnp.sum(a, axis=0, keepdims=True)
    inv_c = jnp.where(d_col > 0, 1.0 / jnp.sqrt(d_col), 0.0)
    inv_r = jnp.where(d_row > 0, 1.0 / jnp.sqrt(d_row), 0.0)
    sup = -(inv_c * a) * inv_r
    sup_ref[...] = sup
    x0 = x_ref[...]
    a1 = jnp.dot(sup, x0, preferred_element_type=jnp.float32)
    a1_ref[...] = a1
    a2_ref[...] = 2.0 * jnp.dot(sup, a1, preferred_element_type=jnp.float32) - x0


def _main_kernel(sup_ref, ac_ref, h0_ref, h1_ref,
                 wa_ru0_ref, wh_ru0_ref, b_ru0_ref,
                 wa_c0_ref, wh_c0_ref, b_c0_ref,
                 wg_ru1_ref, wk_ru1_ref, b_ru1_ref,
                 wg_c1_ref, wk_c1_ref, b_c1_ref,
                 wp_ref, bp_ref,
                 out_ref, h0o_ref, h1o_ref):
    f32 = jnp.float32
    dot = lambda x, y: jnp.dot(x, y, preferred_element_type=f32)
    S = sup_ref[...]
    A = jnp.concatenate([ac_ref[0], ac_ref[1]], axis=1)

    H0 = jnp.concatenate([h0_ref[0], h0_ref[1]], axis=1)
    ru = b_ru0_ref[...] + dot(A, wa_ru0_ref[...])
    ru += dot(H0, wh_ru0_ref[0])
    H1 = dot(S, H0)
    ru += dot(H1, wh_ru0_ref[1])
    H2 = 2.0 * dot(S, H1) - H0
    ru += dot(H2, wh_ru0_ref[2])
    val = jax.nn.sigmoid(ru)
    r = val[:, :2 * U]
    u = val[:, 2 * U:]
    rh = r * H0
    c = b_c0_ref[...] + dot(A, wa_c0_ref[...])
    c += dot(rh, wh_c0_ref[0])
    R1 = dot(S, rh)
    c += dot(R1, wh_c0_ref[1])
    R2 = 2.0 * dot(S, R1) - rh
    c += dot(R2, wh_c0_ref[2])
    c = jnp.tanh(c)
    h0n = u * H0 + (1.0 - u) * c
    h0o_ref[0] = h0n[:, :U]
    h0o_ref[1] = h0n[:, U:]

    K0 = jnp.concatenate([h1_ref[0], h1_ref[1]], axis=1)
    ru1 = b_ru1_ref[...] + dot(h0n, wg_ru1_ref[0]) + dot(K0, wk_ru1_ref[0])
    G1 = dot(S, h0n)
    ru1 += dot(G1, wg_ru1_ref[1])
    K1 = dot(S, K0)
    ru1 += dot(K1, wk_ru1_ref[1])
    G2 = 2.0 * dot(S, G1) - h0n
    ru1 += dot(G2, wg_ru1_ref[2])
    K2 = 2.0 * dot(S, K1) - K0
    ru1 += dot(K2, wk_ru1_ref[2])
    v1 = jax.nn.sigmoid(ru1)
    r1 = v1[:, :2 * U]
    u1 = v1[:, 2 * U:]
    rh1 = r1 * K0
    c1 = (b_c1_ref[...] + dot(h0n, wg_c1_ref[0]) + dot(G1, wg_c1_ref[1])
          + dot(G2, wg_c1_ref[2]) + dot(rh1, wk_c1_ref[0]))
    Q1 = dot(S, rh1)
    c1 += dot(Q1, wk_c1_ref[1])
    Q2 = 2.0 * dot(S, Q1) - rh1
    c1 += dot(Q2, wk_c1_ref[2])
    c1 = jnp.tanh(c1)
    h1n = u1 * K0 + (1.0 - u1) * c1
    h1o_ref[0] = h1n[:, :U]
    h1o_ref[1] = h1n[:, U:]
    prj = dot(h1n, wp_ref[...]) + bp_ref[...]
    out_ref[0] = prj[:, 0:1]
    out_ref[1] = prj[:, 1:2]


def _pair_ru(w):
    wr, wu = w[:, :U], w[:, U:]
    z = jnp.zeros_like(wr)
    top = jnp.concatenate([wr, z, wu, z], axis=1)
    bot = jnp.concatenate([z, wr, z, wu], axis=1)
    return jnp.concatenate([top, bot], axis=0)


def _pair_c(w):
    z = jnp.zeros_like(w)
    return jnp.concatenate([jnp.concatenate([w, z], axis=1),
                            jnp.concatenate([z, w], axis=1)], axis=0)


def kernel(inputs, hidden_state, adj_mx, W_ru_0, b_ru_0, W_c_0, b_c_0,
           W_ru_1, b_ru_1, W_c_1, b_c_1, W_proj, b_proj):
    f32 = jnp.float32
    inp_t = inputs.T
    hs = hidden_state.reshape(2, B, N, U)

    wru0 = W_ru_0.reshape(U + 1, NK, 2 * U)
    wa_ru0 = _pair_ru(wru0[0])
    wh_ru0 = jnp.stack([_pair_ru(wru0[1:, k, :]) for k in range(NK)])
    wc0 = W_c_0.reshape(U + 1, NK, U)
    wa_c0 = _pair_c(wc0[0])
    wh_c0 = jnp.stack([_pair_c(wc0[1:, k, :]) for k in range(NK)])
    wru1 = W_ru_1.reshape(2 * U, NK, 2 * U)
    wg_ru1 = jnp.stack([_pair_ru(wru1[:U, k, :]) for k in range(NK)])
    wk_ru1 = jnp.stack([_pair_ru(wru1[U:, k, :]) for k in range(NK)])
    wc1 = W_c_1.reshape(2 * U, NK, U)
    wg_c1 = jnp.stack([_pair_c(wc1[:U, k, :]) for k in range(NK)])
    wk_c1 = jnp.stack([_pair_c(wc1[U:, k, :]) for k in range(NK)])

    b_ru0 = jnp.concatenate([b_ru_0[:U], b_ru_0[:U], b_ru_0[U:], b_ru_0[U:]]).reshape(1, 4 * U)
    b_c0 = jnp.concatenate([b_c_0, b_c_0]).reshape(1, 2 * U)
    b_ru1 = jnp.concatenate([b_ru_1[:U], b_ru_1[:U], b_ru_1[U:], b_ru_1[U:]]).reshape(1, 4 * U)
    b_c1 = jnp.concatenate([b_c_1, b_c_1]).reshape(1, 2 * U)
    wp = _pair_c(W_proj)
    bp = b_proj.reshape(1, 1)

    support, A1, A2 = pl.pallas_call(
        _prep_kernel,
        out_shape=[
            jax.ShapeDtypeStruct((N, N), f32),
            jax.ShapeDtypeStruct((N, B), f32),
            jax.ShapeDtypeStruct((N, B), f32),
        ],
    )(adj_mx, adj_mx.T, inp_t)

    acat = jnp.stack([inp_t, A1, A2], axis=2).transpose(1, 0, 2)

    full = lambda shape: pl.BlockSpec(shape, lambda b: tuple(0 for _ in shape))
    acol = pl.BlockSpec((2, N, NK), lambda b: (b, 0, 0))
    ocol = pl.BlockSpec((2, N, 1), lambda b: (b, 0, 0))
    hblk = pl.BlockSpec((2, N, U), lambda b: (b, 0, 0))

    out_3, h0o, h1o = pl.pallas_call(
        _main_kernel,
        grid=(B // 2,),
        in_specs=[
            full((N, N)), acol, hblk, hblk,
            full((2 * NK, 4 * U)), full((NK, 2 * U, 4 * U)), full((1, 4 * U)),
            full((2 * NK, 2 * U)), full((NK, 2 * U, 2 * U)), full((1, 2 * U)),
            full((NK, 2 * U, 4 * U)), full((NK, 2 * U, 4 * U)), full((1, 4 * U)),
            full((NK, 2 * U, 2 * U)), full((NK, 2 * U, 2 * U)), full((1, 2 * U)),
            full((2 * U, 2)), full((1, 1)),
        ],
        out_specs=[ocol, hblk, hblk],
        out_shape=[
            jax.ShapeDtypeStruct((B, N, 1), f32),
            jax.ShapeDtypeStruct((B, N, U), f32),
            jax.ShapeDtypeStruct((B, N, U), f32),
        ],
    )(support, acat, hs[0], hs[1],
      wa_ru0, wh_ru0, b_ru0, wa_c0, wh_c0, b_c0,
      wg_ru1, wk_ru1, b_ru1, wg_c1, wk_c1, b_c1,
      wp, bp)

    out = out_3.reshape(B, N)
    hidden = jnp.stack([h0o.reshape(B, N * U), h1o.reshape(B, N * U)])
    return (out, hidden)
